# baseline (speedup 1.0000x reference)
import jax
import jax.numpy as jnp
from jax import lax
from jax.experimental import pallas as pl
from jax.experimental.pallas import tpu as pltpu
from jax.experimental.pallas import tpu_sc as plsc

N = 10000
C = 128
NPAD = 10240
E = 320000
CHUNK = 80
NCORES = 2
NSUB = 16
ROWS_PER_SUB = NPAD // NSUB
EDGES_PER_CORE = E // NCORES
EDGES_PER_SUB = EDGES_PER_CORE // NSUB
N_CHUNKS = EDGES_PER_SUB // CHUNK

_MESH = plsc.VectorSubcoreMesh(core_axis_name="c", subcore_axis_name="s")


def _deg_body(dst_hbm, zeros_hbm, ones_hbm, deg_out, deg_sh, ones_v, idx_v,
              si, ss):
    cid = lax.axis_index("c")
    sid = lax.axis_index("s")
    row0 = sid * ROWS_PER_SUB
    pltpu.sync_copy(zeros_hbm.at[pl.ds(row0, ROWS_PER_SUB)],
                    deg_sh.at[pl.ds(row0, ROWS_PER_SUB)])
    pltpu.sync_copy(ones_hbm, ones_v)
    plsc.subcore_barrier()

    ebase = cid * EDGES_PER_CORE + sid * EDGES_PER_SUB

    def start_idx(g, base):
        pltpu.async_copy(dst_hbm.at[pl.ds(base, CHUNK)], idx_v.at[g % 8],
                         si.at[g % 8])

    def wait_idx(g):
        pltpu.make_async_copy(dst_hbm.at[pl.ds(0, CHUNK)],
                              idx_v.at[g % 8], si.at[g % 8]).wait()

    def start_scatter(g):
        pltpu.async_copy(ones_v, deg_sh.at[idx_v.at[g % 8]], ss.at[g % 8],
                         add=True)

    def wait_scatter(g):
        pltpu.make_async_copy(ones_v, deg_sh.at[pl.ds(0, CHUNK)],
                              ss.at[g % 8]).wait()

    def step(g, base_idx):
        wait_idx(g)
        start_scatter(g)
        if g >= 4:
            wait_scatter(g - 4)
        if g + 4 < N_CHUNKS:
            start_idx(g + 4, base_idx)

    for g in range(4):
        start_idx(g, ebase + g * CHUNK)
    for g in range(8):
        step(g, ebase + (g + 4) * CHUNK)

    _K = (N_CHUNKS - 16) // 8

    def body(gg, carry):
        off = ebase + gg * 8 * CHUNK
        for j in range(8):
            g = 8 + j
            step(g, off + (g + 4) * CHUNK)
        return carry

    lax.fori_loop(0, _K, body, 0)

    for g in range(8 + _K * 8, N_CHUNKS):
        step(g, ebase + (g + 4) * CHUNK)
    for g in range(N_CHUNKS - 4, N_CHUNKS):
        wait_scatter(g)

    plsc.subcore_barrier()
    pltpu.sync_copy(deg_sh.at[pl.ds(row0, ROWS_PER_SUB)],
                    deg_out.at[cid, pl.ds(row0, ROWS_PER_SUB)])


_k_deg = pl.kernel(
    _deg_body,
    out_type=jax.ShapeDtypeStruct((NCORES, NPAD), jnp.float32),
    mesh=_MESH,
    scratch_types=[
        pltpu.VMEM_SHARED((NPAD,), jnp.float32),
        pltpu.VMEM((CHUNK,), jnp.float32),
        pltpu.VMEM((8, CHUNK), jnp.int32),
        pltpu.SemaphoreType.DMA((8,)),
        pltpu.SemaphoreType.DMA((8,)),
    ],
)


PCHUNK = CHUNK
N_PCH = EDGES_PER_SUB // PCHUNK
PROP_RB = 4
PROP_IB = 8
PROP_L = 3
PROP_S = 1
PROP_I = 4
PROP_UNROLL = 8


def _prop_body(tab_hbm, src_hbm, dst_hbm, acc_out,
               acc_sh, src_v, dst_v, rows_v, si, sg, ss):
    cid = lax.axis_index("c")
    sid = lax.axis_index("s")
    row0 = sid * ROWS_PER_SUB
    pltpu.sync_copy(tab_hbm.at[pl.ds(row0, ROWS_PER_SUB)],
                    acc_sh.at[pl.ds(row0, ROWS_PER_SUB)])
    plsc.subcore_barrier()

    ebase = cid * EDGES_PER_CORE + sid * EDGES_PER_SUB

    def start_idx(g, base):
        b = g % PROP_IB
        pltpu.async_copy(src_hbm.at[pl.ds(base, PCHUNK)], src_v.at[b],
                         si.at[b])
        pltpu.async_copy(dst_hbm.at[pl.ds(base, PCHUNK)], dst_v.at[b],
                         si.at[b])

    def wait_idx(g):
        b = g % PROP_IB
        pltpu.make_async_copy(src_hbm.at[pl.ds(0, PCHUNK)], src_v.at[b],
                              si.at[b]).wait()
        pltpu.make_async_copy(dst_hbm.at[pl.ds(0, PCHUNK)], dst_v.at[b],
                              si.at[b]).wait()

    def start_gather(g):
        pltpu.async_copy(tab_hbm.at[src_v.at[g % PROP_IB]],
                         rows_v.at[g % PROP_RB], sg.at[g % PROP_RB])

    def wait_gather(g):
        b = g % PROP_RB
        pltpu.make_async_copy(tab_hbm.at[pl.ds(0, PCHUNK)], rows_v.at[b],
                              sg.at[b]).wait()

    def start_scatter(g):
        pltpu.async_copy(rows_v.at[g % PROP_RB],
                         acc_sh.at[dst_v.at[g % PROP_IB]],
                         ss.at[g % PROP_RB], add=True)

    def wait_scatter(g):
        b = g % PROP_RB
        pltpu.make_async_copy(rows_v.at[b], acc_sh.at[pl.ds(0, PCHUNK)],
                              ss.at[b]).wait()

    def step(g, base_idx):
        wait_gather(g)
        start_scatter(g)
        if g >= PROP_S:
            wait_scatter(g - PROP_S)
        if g + PROP_L < N_PCH:
            wait_idx(g + PROP_L)
            start_gather(g + PROP_L)
        if g + PROP_I < N_PCH:
            start_idx(g + PROP_I, base_idx)

    for g in range(PROP_I):
        start_idx(g, ebase + g * PCHUNK)
    for g in range(PROP_L):
        wait_idx(g)
        start_gather(g)

    for g in range(PROP_UNROLL):
        step(g, ebase + (g + PROP_I) * PCHUNK)

    _K = (N_PCH - 2 * PROP_UNROLL) // PROP_UNROLL

    def body(gg, carry):
        off = ebase + gg * PROP_UNROLL * PCHUNK
        for j in range(PROP_UNROLL):
            g = PROP_UNROLL + j
            step(g, off + (g + PROP_I) * PCHUNK)
        return carry

    lax.fori_loop(0, _K, body, 0)

    for g in range(PROP_UNROLL + _K * PROP_UNROLL, N_PCH):
        step(g, ebase + (g + PROP_I) * PCHUNK)
    for g in range(N_PCH - PROP_S, N_PCH):
        wait_scatter(g)

    plsc.subcore_barrier()
    pltpu.sync_copy(acc_sh.at[pl.ds(row0, ROWS_PER_SUB)],
                    acc_out.at[cid, pl.ds(row0, ROWS_PER_SUB)])


_k_prop = pl.kernel(
    _prop_body,
    out_type=jax.ShapeDtypeStruct((NCORES, NPAD, C), jnp.float32),
    mesh=_MESH,
    scratch_types=[
        pltpu.VMEM_SHARED((NPAD, C), jnp.float32),
        pltpu.VMEM((PROP_IB, PCHUNK), jnp.int32),
        pltpu.VMEM((PROP_IB, PCHUNK), jnp.int32),
        pltpu.VMEM((PROP_RB, PCHUNK, C), jnp.float32),
        pltpu.SemaphoreType.DMA((PROP_IB,)),
        pltpu.SemaphoreType.DMA((PROP_RB,)),
        pltpu.SemaphoreType.DMA((PROP_RB,)),
    ],
)


BLK = 1024
GRID = NPAD // BLK


def _dinv_block(degs_ref, i):
    sl = pl.ds(i * BLK, BLK)
    deg = degs_ref[0, sl] + degs_ref[1, sl]
    row = lax.broadcasted_iota(jnp.int32, (BLK,), 0) + i * BLK
    return jnp.where(row < N, lax.rsqrt(deg + 1.0), 0.0)[:, None]


def _mm_body(x_ref, w1_ref, xw_ref):
    xw_ref[...] = jnp.dot(x_ref[...], w1_ref[...],
                          preferred_element_type=jnp.float32)


def _scale_body(xw_ref, degs_ref, y1_ref):
    dinv = _dinv_block(degs_ref, pl.program_id(0))
    y1_ref[...] = xw_ref[...] * dinv


def _y2_body(acc_ref, y1_ref, degs_ref, b1_ref, w2_ref, y2_ref):
    dinv = _dinv_block(degs_ref, pl.program_id(0))
    a = acc_ref[0] + acc_ref[1] - y1_ref[...]
    h = jnp.maximum(a * dinv + b1_ref[...], 0.0)
    y2_ref[...] = jnp.dot(h, w2_ref[...],
                          preferred_element_type=jnp.float32) * dinv


def _out_body(acc_ref, y2_ref, degs_ref, b2_ref, out_ref):
    dinv = _dinv_block(degs_ref, pl.program_id(0))
    a = acc_ref[0] + acc_ref[1] - y2_ref[...]
    out_ref[...] = a * dinv + b2_ref[...]


_full_degs = pl.BlockSpec((NCORES, NPAD), lambda i: (0, 0))
_full_w = pl.BlockSpec((C, C), lambda i: (0, 0))
_full_b = pl.BlockSpec((1, C), lambda i: (0, 0))
_row_blk = pl.BlockSpec((BLK, C), lambda i: (i, 0))
_acc_blk = pl.BlockSpec((NCORES, BLK, C), lambda i: (0, i, 0))

_k_mm = pl.pallas_call(
    _mm_body,
    grid=(GRID,),
    in_specs=[_row_blk, _full_w],
    out_specs=_row_blk,
    out_shape=jax.ShapeDtypeStruct((NPAD, C), jnp.float32),
)

_k_scale = pl.pallas_call(
    _scale_body,
    grid=(GRID,),
    in_specs=[_row_blk, _full_degs],
    out_specs=_row_blk,
    out_shape=jax.ShapeDtypeStruct((NPAD, C), jnp.float32),
)

_k_y2 = pl.pallas_call(
    _y2_body,
    grid=(GRID,),
    in_specs=[_acc_blk, _row_blk, _full_degs, _full_b, _full_w],
    out_specs=_row_blk,
    out_shape=jax.ShapeDtypeStruct((NPAD, C), jnp.float32),
)

_k_out = pl.pallas_call(
    _out_body,
    grid=(GRID,),
    in_specs=[_acc_blk, _row_blk, _full_degs, _full_b],
    out_specs=_row_blk,
    out_shape=jax.ShapeDtypeStruct((N, C), jnp.float32),
)


def kernel(x, adj, W1, b1, W2, b2):
    src = adj[0].astype(jnp.int32)
    dst = adj[1].astype(jnp.int32)
    xp = jnp.pad(x, ((0, NPAD - N), (0, 0)))
    zeros = jnp.zeros((NPAD,), jnp.float32)
    ones = jnp.ones((CHUNK,), jnp.float32)
    b1r = b1.reshape(1, C)
    b2r = b2.reshape(1, C)

    xw1 = _k_mm(xp, W1)
    degs = _k_deg(dst, zeros, ones)
    y1 = _k_scale(xw1, degs)
    acc1 = _k_prop(y1, src, dst)
    y2 = _k_y2(acc1, y1, degs, b1r, W2)
    acc2 = _k_prop(y2, src, dst)
    return _k_out(acc2, y2, degs, b2r)


# speedup vs baseline: 1.0046x; 1.0046x over previous
import jax
import jax.numpy as jnp
from jax import lax
from jax.experimental import pallas as pl
from jax.experimental.pallas import tpu as pltpu
from jax.experimental.pallas import tpu_sc as plsc

N = 10000
C = 128
NPAD = 10240
E = 320000
CHUNK = 80
NCORES = 2
NSUB = 16
ROWS_PER_SUB = NPAD // NSUB
EDGES_PER_CORE = E // NCORES
EDGES_PER_SUB = EDGES_PER_CORE // NSUB
N_CHUNKS = EDGES_PER_SUB // CHUNK

_MESH = plsc.VectorSubcoreMesh(core_axis_name="c", subcore_axis_name="s")


def _deg_body(dst_hbm, zeros_hbm, ones_hbm, deg_out, deg_sh, ones_v, idx_v,
              si, ss):
    cid = lax.axis_index("c")
    sid = lax.axis_index("s")
    row0 = sid * ROWS_PER_SUB
    pltpu.sync_copy(zeros_hbm.at[pl.ds(row0, ROWS_PER_SUB)],
                    deg_sh.at[pl.ds(row0, ROWS_PER_SUB)])
    pltpu.sync_copy(ones_hbm, ones_v)
    plsc.subcore_barrier()

    ebase = cid * EDGES_PER_CORE + sid * EDGES_PER_SUB

    def start_idx(g, base):
        pltpu.async_copy(dst_hbm.at[pl.ds(base, CHUNK)], idx_v.at[g % 8],
                         si.at[g % 8])

    def wait_idx(g):
        pltpu.make_async_copy(dst_hbm.at[pl.ds(0, CHUNK)],
                              idx_v.at[g % 8], si.at[g % 8]).wait()

    def start_scatter(g):
        pltpu.async_copy(ones_v, deg_sh.at[idx_v.at[g % 8]], ss.at[g % 8],
                         add=True)

    def wait_scatter(g):
        pltpu.make_async_copy(ones_v, deg_sh.at[pl.ds(0, CHUNK)],
                              ss.at[g % 8]).wait()

    def step(g, base_idx):
        wait_idx(g)
        start_scatter(g)
        if g >= 4:
            wait_scatter(g - 4)
        if g + 4 < N_CHUNKS:
            start_idx(g + 4, base_idx)

    for g in range(4):
        start_idx(g, ebase + g * CHUNK)
    for g in range(8):
        step(g, ebase + (g + 4) * CHUNK)

    _K = (N_CHUNKS - 16) // 8

    def body(gg, carry):
        off = ebase + gg * 8 * CHUNK
        for j in range(8):
            g = 8 + j
            step(g, off + (g + 4) * CHUNK)
        return carry

    lax.fori_loop(0, _K, body, 0)

    for g in range(8 + _K * 8, N_CHUNKS):
        step(g, ebase + (g + 4) * CHUNK)
    for g in range(N_CHUNKS - 4, N_CHUNKS):
        wait_scatter(g)

    plsc.subcore_barrier()
    pltpu.sync_copy(deg_sh.at[pl.ds(row0, ROWS_PER_SUB)],
                    deg_out.at[cid, pl.ds(row0, ROWS_PER_SUB)])


_k_deg = pl.kernel(
    _deg_body,
    out_type=jax.ShapeDtypeStruct((NCORES, NPAD), jnp.float32),
    mesh=_MESH,
    scratch_types=[
        pltpu.VMEM_SHARED((NPAD,), jnp.float32),
        pltpu.VMEM((CHUNK,), jnp.float32),
        pltpu.VMEM((8, CHUNK), jnp.int32),
        pltpu.SemaphoreType.DMA((8,)),
        pltpu.SemaphoreType.DMA((8,)),
    ],
)


PCHUNK = CHUNK
N_PCH = EDGES_PER_SUB // PCHUNK
PROP_RB = 4
PROP_IB = 8
PROP_L = 3
PROP_S = 1
PROP_I = 4
PROP_UNROLL = 8


def _prop_body(tab_hbm, src_hbm, dst_hbm, acc_out,
               acc_sh, src_v, dst_v, rows_v, si, sg, ss):
    cid = lax.axis_index("c")
    sid = lax.axis_index("s")
    row0 = sid * ROWS_PER_SUB
    pltpu.sync_copy(tab_hbm.at[pl.ds(row0, ROWS_PER_SUB)],
                    acc_sh.at[pl.ds(row0, ROWS_PER_SUB)])
    plsc.subcore_barrier()

    ebase = cid * EDGES_PER_CORE + sid * EDGES_PER_SUB

    def start_idx(g, base):
        b = g % PROP_IB
        pltpu.async_copy(src_hbm.at[pl.ds(base, PCHUNK)], src_v.at[b],
                         si.at[b])
        pltpu.async_copy(dst_hbm.at[pl.ds(base, PCHUNK)], dst_v.at[b],
                         si.at[b])

    def wait_idx(g):
        b = g % PROP_IB
        pltpu.make_async_copy(src_hbm.at[pl.ds(0, PCHUNK)], src_v.at[b],
                              si.at[b]).wait()
        pltpu.make_async_copy(dst_hbm.at[pl.ds(0, PCHUNK)], dst_v.at[b],
                              si.at[b]).wait()

    def start_gather(g):
        pltpu.async_copy(tab_hbm.at[src_v.at[g % PROP_IB]],
                         rows_v.at[g % PROP_RB], sg.at[g % PROP_RB])

    def wait_gather(g):
        b = g % PROP_RB
        pltpu.make_async_copy(tab_hbm.at[pl.ds(0, PCHUNK)], rows_v.at[b],
                              sg.at[b]).wait()

    def start_scatter(g):
        pltpu.async_copy(rows_v.at[g % PROP_RB],
                         acc_sh.at[dst_v.at[g % PROP_IB]],
                         ss.at[g % PROP_RB], add=True)

    def wait_scatter(g):
        b = g % PROP_RB
        pltpu.make_async_copy(rows_v.at[b], acc_sh.at[pl.ds(0, PCHUNK)],
                              ss.at[b]).wait()

    def step(g, base_idx):
        wait_gather(g)
        start_scatter(g)
        if g >= PROP_S:
            wait_scatter(g - PROP_S)
        if g + PROP_L < N_PCH:
            wait_idx(g + PROP_L)
            start_gather(g + PROP_L)
        if g + PROP_I < N_PCH:
            start_idx(g + PROP_I, base_idx)

    for g in range(PROP_I):
        start_idx(g, ebase + g * PCHUNK)
    for g in range(PROP_L):
        wait_idx(g)
        start_gather(g)

    for g in range(PROP_UNROLL):
        step(g, ebase + (g + PROP_I) * PCHUNK)

    _K = (N_PCH - 2 * PROP_UNROLL) // PROP_UNROLL

    def body(gg, carry):
        off = ebase + gg * PROP_UNROLL * PCHUNK
        for j in range(PROP_UNROLL):
            g = PROP_UNROLL + j
            step(g, off + (g + PROP_I) * PCHUNK)
        return carry

    lax.fori_loop(0, _K, body, 0)

    for g in range(PROP_UNROLL + _K * PROP_UNROLL, N_PCH):
        step(g, ebase + (g + PROP_I) * PCHUNK)
    for g in range(N_PCH - PROP_S, N_PCH):
        wait_scatter(g)

    plsc.subcore_barrier()
    pltpu.sync_copy(acc_sh.at[pl.ds(row0, ROWS_PER_SUB)],
                    acc_out.at[cid, pl.ds(row0, ROWS_PER_SUB)])


_k_prop = pl.kernel(
    _prop_body,
    out_type=jax.ShapeDtypeStruct((NCORES, NPAD, C), jnp.float32),
    mesh=_MESH,
    scratch_types=[
        pltpu.VMEM_SHARED((NPAD, C), jnp.float32),
        pltpu.VMEM((PROP_IB, PCHUNK), jnp.int32),
        pltpu.VMEM((PROP_IB, PCHUNK), jnp.int32),
        pltpu.VMEM((PROP_RB, PCHUNK, C), jnp.float32),
        pltpu.SemaphoreType.DMA((PROP_IB,)),
        pltpu.SemaphoreType.DMA((PROP_RB,)),
        pltpu.SemaphoreType.DMA((PROP_RB,)),
    ],
)


BLK = 1024
GRID = NPAD // BLK


def _dinv_block(degs_ref, i):
    sl = pl.ds(i * BLK, BLK)
    deg = degs_ref[0, sl] + degs_ref[1, sl]
    row = lax.broadcasted_iota(jnp.int32, (BLK,), 0) + i * BLK
    return jnp.where(row < N, lax.rsqrt(deg + 1.0), 0.0)[:, None]


def _y1_body(x_ref, w1_ref, degs_ref, y1_ref):
    dinv = _dinv_block(degs_ref, pl.program_id(0))
    y1_ref[...] = jnp.dot(x_ref[...], w1_ref[...],
                          preferred_element_type=jnp.float32) * dinv


def _y2_body(acc_ref, y1_ref, degs_ref, b1_ref, w2_ref, y2_ref):
    dinv = _dinv_block(degs_ref, pl.program_id(0))
    a = acc_ref[0] + acc_ref[1] - y1_ref[...]
    h = jnp.maximum(a * dinv + b1_ref[...], 0.0)
    y2_ref[...] = jnp.dot(h, w2_ref[...],
                          preferred_element_type=jnp.float32) * dinv


def _out_body(acc_ref, y2_ref, degs_ref, b2_ref, out_ref):
    dinv = _dinv_block(degs_ref, pl.program_id(0))
    a = acc_ref[0] + acc_ref[1] - y2_ref[...]
    out_ref[...] = a * dinv + b2_ref[...]


_full_degs = pl.BlockSpec((NCORES, NPAD), lambda i: (0, 0))
_full_w = pl.BlockSpec((C, C), lambda i: (0, 0))
_full_b = pl.BlockSpec((1, C), lambda i: (0, 0))
_row_blk = pl.BlockSpec((BLK, C), lambda i: (i, 0))
_acc_blk = pl.BlockSpec((NCORES, BLK, C), lambda i: (0, i, 0))

_k_y1 = pl.pallas_call(
    _y1_body,
    grid=(GRID,),
    in_specs=[_row_blk, _full_w, _full_degs],
    out_specs=_row_blk,
    out_shape=jax.ShapeDtypeStruct((NPAD, C), jnp.float32),
)

_k_y2 = pl.pallas_call(
    _y2_body,
    grid=(GRID,),
    in_specs=[_acc_blk, _row_blk, _full_degs, _full_b, _full_w],
    out_specs=_row_blk,
    out_shape=jax.ShapeDtypeStruct((NPAD, C), jnp.float32),
)

_k_out = pl.pallas_call(
    _out_body,
    grid=(GRID,),
    in_specs=[_acc_blk, _row_blk, _full_degs, _full_b],
    out_specs=_row_blk,
    out_shape=jax.ShapeDtypeStruct((N, C), jnp.float32),
)


def kernel(x, adj, W1, b1, W2, b2):
    src = adj[0].astype(jnp.int32)
    dst = adj[1].astype(jnp.int32)
    xp = jnp.pad(x, ((0, NPAD - N), (0, 0)))
    zeros = jnp.zeros((NPAD,), jnp.float32)
    ones = jnp.ones((CHUNK,), jnp.float32)
    b1r = b1.reshape(1, C)
    b2r = b2.reshape(1, C)

    degs = _k_deg(dst, zeros, ones)
    y1 = _k_y1(xp, W1, degs)
    acc1 = _k_prop(y1, src, dst)
    y2 = _k_y2(acc1, y1, degs, b1r, W2)
    acc2 = _k_prop(y2, src, dst)
    return _k_out(acc2, y2, degs, b2r)
